# baseline (speedup 1.0000x reference)
import functools

import jax
import jax.numpy as jnp
from jax import lax
from jax.experimental import pallas as pl
from jax.experimental.pallas import tpu as pltpu
from jax.experimental.pallas import tpu_sc as plsc

K = 8
SC_SLABS = 64


def _sc_partial_kernel(x_hbm, out_hbm, buf0, buf1, acc, sem0, sem1,
                       *, s0, s1, bW, dim):
    wid = lax.axis_index("s") * 2 + lax.axis_index("c")
    base = wid * bW
    bufs = (buf0, buf1)
    sems = (sem0, sem1)

    pltpu.make_async_copy(x_hbm.at[s0, pl.ds(base, bW), :], buf0, sem0).start()
    pltpu.make_async_copy(x_hbm.at[s0 + 1, pl.ds(base, bW), :], buf1,
                          sem1).start()

    zero = jnp.zeros((16,), jnp.float32)

    @pl.loop(0, bW)
    def _zero_row(r):
        for j in range(dim // 16):
            acc[r, pl.ds(j * 16, 16)] = zero

    @pl.loop(0, s1 - s0, step=2)
    def _slab_pair(i):
        for b in range(2):
            s = s0 + i + b
            pltpu.make_async_copy(x_hbm.at[s, pl.ds(base, bW), :], bufs[b],
                                  sems[b]).wait()

            @pl.loop(0, bW)
            def _row(r):
                for j in range(dim // 16):
                    sl = pl.ds(j * 16, 16)
                    plsc.addupdate(acc.at[r, sl], bufs[b][r, sl])

            @pl.when(s + 2 < s1)
            def _start_next():
                pltpu.make_async_copy(x_hbm.at[s + 2, pl.ds(base, bW), :],
                                      bufs[b], sems[b]).start()

    pltpu.sync_copy(acc, out_hbm.at[pl.ds(base, bW), :])


def _tc_reduce_kernel(x_ref, psum_ref):
    psum_ref[...] = jnp.sum(x_ref[...], axis=0)


def _router_kernel(tcp_ref, scp_ref, freq_ref, noise_ref, gw_ref, fgw_ref,
                   gates_ref, idx_ref, val_ref, *, spatial):
    pooled = (tcp_ref[...] + scp_ref[...]) * (1.0 / spatial)
    logits = (
        jax.lax.dot(pooled, gw_ref[...], preferred_element_type=jnp.float32)
        + jax.lax.dot(freq_ref[...], fgw_ref[...],
                      preferred_element_type=jnp.float32)
        + noise_ref[...]
    )
    m = jnp.max(logits, axis=1, keepdims=True)
    e = jnp.exp(logits - m)
    probs = e / jnp.sum(e, axis=1, keepdims=True)

    bB, E = probs.shape
    lane = jax.lax.broadcasted_iota(jnp.int32, (bB, E), 1)
    work = probs
    keep = jnp.zeros((bB, E), dtype=jnp.bool_)
    vals = []
    idxs = []
    for _ in range(K):
        cur = jnp.max(work, axis=1, keepdims=True)
        cur_i = jnp.min(jnp.where(work == cur, lane, E), axis=1,
                        keepdims=True)
        sel = lane == cur_i
        keep = jnp.logical_or(keep, sel)
        work = jnp.where(sel, -jnp.inf, work)
        vals.append(cur)
        idxs.append(cur_i)
    gates_ref[...] = jnp.where(keep, probs, 0.0)
    val_ref[...] = jnp.concatenate(vals, axis=1)
    idx_ref[...] = jnp.concatenate(idxs, axis=1)


def kernel(x, freq_emb, gate_w, freq_gate_w):
    B, DIM, H, W = x.shape
    FREQ = freq_emb.shape[1]
    E = gate_w.shape[0]
    S = H * W
    S_TC = S - SC_SLABS
    noise_std = 1.0 / E
    noise = jax.random.normal(jax.random.key(42), (B, E),
                              dtype=jnp.float32) * noise_std

    x_t = x.transpose(2, 3, 0, 1).reshape(S, B, DIM)
    gw_t = gate_w.T
    fgw_t = freq_gate_w.T

    bW = B // 32
    sc_partial = pl.kernel(
        functools.partial(_sc_partial_kernel, s0=S_TC, s1=S, bW=bW, dim=DIM),
        out_type=jax.ShapeDtypeStruct((B, DIM), jnp.float32),
        mesh=plsc.VectorSubcoreMesh(core_axis_name="c", subcore_axis_name="s"),
        scratch_types=[
            pltpu.VMEM((bW, DIM), jnp.float32),
            pltpu.VMEM((bW, DIM), jnp.float32),
            pltpu.VMEM((bW, DIM), jnp.float32),
            pltpu.SemaphoreType.DMA,
            pltpu.SemaphoreType.DMA,
        ],
    )(x_t)

    bB = 128
    tc_partial = pl.pallas_call(
        _tc_reduce_kernel,
        grid=(B // bB,),
        in_specs=[pl.BlockSpec((S_TC, bB, DIM), lambda i: (0, i, 0))],
        out_specs=pl.BlockSpec((bB, DIM), lambda i: (i, 0)),
        out_shape=jax.ShapeDtypeStruct((B, DIM), jnp.float32),
        compiler_params=pltpu.CompilerParams(
            dimension_semantics=("arbitrary",),
        ),
    )(x_t)

    bB2 = 256
    gates, idxs, vals = pl.pallas_call(
        functools.partial(_router_kernel, spatial=float(S)),
        grid=(B // bB2,),
        in_specs=[
            pl.BlockSpec((bB2, DIM), lambda i: (i, 0)),
            pl.BlockSpec((bB2, DIM), lambda i: (i, 0)),
            pl.BlockSpec((bB2, FREQ), lambda i: (i, 0)),
            pl.BlockSpec((bB2, E), lambda i: (i, 0)),
            pl.BlockSpec((DIM, E), lambda i: (0, 0)),
            pl.BlockSpec((FREQ, E), lambda i: (0, 0)),
        ],
        out_specs=[
            pl.BlockSpec((bB2, E), lambda i: (i, 0)),
            pl.BlockSpec((bB2, K), lambda i: (i, 0)),
            pl.BlockSpec((bB2, K), lambda i: (i, 0)),
        ],
        out_shape=[
            jax.ShapeDtypeStruct((B, E), jnp.float32),
            jax.ShapeDtypeStruct((B, K), jnp.int32),
            jax.ShapeDtypeStruct((B, K), jnp.float32),
        ],
        compiler_params=pltpu.CompilerParams(
            dimension_semantics=("arbitrary",),
        ),
    )(tc_partial, sc_partial, freq_emb, noise, gw_t, fgw_t)

    return (gates, idxs, vals, jnp.float32(0.0))
